# Initial kernel scaffold; baseline (speedup 1.0000x reference)
import jax
import jax.numpy as jnp
from jax.experimental import pallas as pl


def kernel(composition_features, radial_spectrum, spex_features, sum_indices):
    raise NotImplementedError("write your pallas kernel here")


# speedup vs baseline: 1.1201x; 1.1201x over previous
import functools

import jax
import jax.numpy as jnp
from jax import lax
from jax.experimental import pallas as pl
from jax.experimental.pallas import tpu as pltpu
from jax.experimental.pallas import tpu_sc as plsc

N_STRUCT = 512
N_SPEC = 4
N_BUCKETS = N_STRUCT * N_SPEC
N_ATOMS = 51200
C_COMP, C_RAD, C_SPEX = 1, 128, 1024

NC, NS, L = 2, 16, 16
CHUNK = 64
ATOMS_PER_TILE = N_ATOMS // NS
CHUNKS_PER_TILE = ATOMS_PER_TILE // CHUNK
S_HALF = C_SPEX // NC
R_HALF = C_RAD // NC
ROWS_PER_TILE = N_BUCKETS // NS


def _body(comp_hbm, rad_hbm, spex_hbm, idx_hbm,
          out_c, out_r, out_s,
          acc_c, acc_r, acc_s,
          cbuf, rbuf, sbuf, idx_v):
    cid = lax.axis_index("c")
    sid = lax.axis_index("s")

    zeros = jnp.zeros((L,), jnp.float32)

    def zrow_s(i, _):
        for j in range(S_HALF // L):
            sbuf[i, pl.ds(j * L, L)] = zeros
        return 0

    lax.fori_loop(0, CHUNK, zrow_s, 0)

    def zrow_r(i, _):
        for j in range(R_HALF // L):
            rbuf[i, pl.ds(j * L, L)] = zeros
        return 0

    lax.fori_loop(0, CHUNK, zrow_r, 0)

    for j in range(CHUNK // L):
        cbuf[pl.ds(j * L, L)] = zeros

    for b in range(ROWS_PER_TILE // CHUNK):
        rows = pl.ds(sid * ROWS_PER_TILE + b * CHUNK, CHUNK)
        pltpu.sync_copy(sbuf, acc_s.at[rows])
        pltpu.sync_copy(rbuf, acc_r.at[rows])
        pltpu.sync_copy(cbuf, acc_c.at[rows])
    plsc.subcore_barrier()

    def chunk_step(k, _):
        a0 = sid * ATOMS_PER_TILE + k * CHUNK
        atoms = pl.ds(a0, CHUNK)
        pltpu.sync_copy(idx_hbm.at[atoms], idx_v)
        pltpu.sync_copy(spex_hbm.at[atoms, pl.ds(cid * S_HALF, S_HALF)], sbuf)
        pltpu.sync_copy(rad_hbm.at[atoms, pl.ds(cid * R_HALF, R_HALF)], rbuf)
        pltpu.sync_copy(sbuf, acc_s.at[idx_v], add=True)
        pltpu.sync_copy(rbuf, acc_r.at[idx_v], add=True)

        @pl.when(cid == 0)
        def _():
            pltpu.sync_copy(comp_hbm.at[atoms], cbuf)
            pltpu.sync_copy(cbuf, acc_c.at[idx_v], add=True)

        return 0

    lax.fori_loop(0, CHUNKS_PER_TILE, chunk_step, 0)
    plsc.subcore_barrier()

    for b in range(ROWS_PER_TILE // CHUNK):
        rows = pl.ds(sid * ROWS_PER_TILE + b * CHUNK, CHUNK)
        pltpu.sync_copy(acc_s.at[rows], sbuf)
        pltpu.sync_copy(sbuf, out_s.at[cid, rows])
        pltpu.sync_copy(acc_r.at[rows], rbuf)
        pltpu.sync_copy(rbuf, out_r.at[cid, rows])

        @pl.when(cid == 0)
        def _():
            pltpu.sync_copy(acc_c.at[rows], cbuf)
            pltpu.sync_copy(cbuf, out_c.at[rows])


@jax.jit
def _segsum(comp, rad, spex, idx):
    mesh = plsc.VectorSubcoreMesh(
        core_axis_name="c", subcore_axis_name="s", num_cores=NC, num_subcores=NS
    )
    out_c, out_r, out_s = pl.kernel(
        _body,
        out_type=[
            jax.ShapeDtypeStruct((N_BUCKETS,), jnp.float32),
            jax.ShapeDtypeStruct((NC, N_BUCKETS, R_HALF), jnp.float32),
            jax.ShapeDtypeStruct((NC, N_BUCKETS, S_HALF), jnp.float32),
        ],
        mesh=mesh,
        compiler_params=pltpu.CompilerParams(use_tc_tiling_on_sc=False),
        scratch_types=[
            pltpu.VMEM_SHARED((N_BUCKETS,), jnp.float32),
            pltpu.VMEM_SHARED((N_BUCKETS, R_HALF), jnp.float32),
            pltpu.VMEM_SHARED((N_BUCKETS, S_HALF), jnp.float32),
            pltpu.VMEM((CHUNK,), jnp.float32),
            pltpu.VMEM((CHUNK, R_HALF), jnp.float32),
            pltpu.VMEM((CHUNK, S_HALF), jnp.float32),
            pltpu.VMEM((CHUNK,), jnp.int32),
        ],
    )(comp.reshape(N_ATOMS), rad, spex, idx)

    rad_full = jnp.concatenate([out_r[0], out_r[1]], axis=1)
    spex_full = jnp.concatenate([out_s[0], out_s[1]], axis=1)
    return jnp.concatenate(
        [
            out_c.reshape(N_STRUCT, N_SPEC),
            rad_full.reshape(N_STRUCT, N_SPEC * C_RAD),
            spex_full.reshape(N_STRUCT, N_SPEC * C_SPEX),
        ],
        axis=1,
    )


def kernel(composition_features, radial_spectrum, spex_features, sum_indices):
    idx = sum_indices.astype(jnp.int32)
    return _segsum(composition_features, radial_spectrum, spex_features, idx)


# speedup vs baseline: 1.1624x; 1.0378x over previous
import jax
import jax.numpy as jnp
from jax import lax
from jax.experimental import pallas as pl
from jax.experimental.pallas import tpu as pltpu
from jax.experimental.pallas import tpu_sc as plsc

N_STRUCT = 512
N_SPEC = 4
N_BUCKETS = N_STRUCT * N_SPEC
N_ATOMS = 51200
C_COMP, C_RAD, C_SPEX = 1, 128, 1024

NC, NS, L = 2, 16, 16
CHUNK = 64
ATOMS_PER_TILE = N_ATOMS // NS
CHUNKS_PER_TILE = ATOMS_PER_TILE // CHUNK
SG = C_SPEX // 128 // NC
R_HALF = C_RAD // NC
ROWS_PER_TILE = N_BUCKETS // NS


def _body(comp_hbm, rad_hbm, spex_hbm, idx_hbm,
          out_c, out_r, out_s,
          acc_c, acc_r, acc_s,
          cbuf, rbuf, sbuf, idx_v):
    cid = lax.axis_index("c")
    sid = lax.axis_index("s")

    zeros = jnp.zeros((L,), jnp.float32)

    def zrow_s(i, _):
        for g in range(SG):
            for j in range(128 // L):
                sbuf[i, g, pl.ds(j * L, L)] = zeros
        return 0

    lax.fori_loop(0, CHUNK, zrow_s, 0)

    def zrow_r(i, _):
        for j in range(R_HALF // L):
            rbuf[i, pl.ds(j * L, L)] = zeros
        return 0

    lax.fori_loop(0, CHUNK, zrow_r, 0)

    for j in range(CHUNK // L):
        cbuf[pl.ds(j * L, L)] = zeros

    for b in range(ROWS_PER_TILE // CHUNK):
        rows = pl.ds(sid * ROWS_PER_TILE + b * CHUNK, CHUNK)
        pltpu.sync_copy(sbuf, acc_s.at[rows])
        pltpu.sync_copy(rbuf, acc_r.at[rows])
        pltpu.sync_copy(cbuf, acc_c.at[rows])
    plsc.subcore_barrier()

    def chunk_step(k, _):
        a0 = sid * ATOMS_PER_TILE + k * CHUNK
        atoms = pl.ds(a0, CHUNK)
        pltpu.sync_copy(idx_hbm.at[atoms], idx_v)
        pltpu.sync_copy(spex_hbm.at[atoms, pl.ds(cid * SG, SG), :], sbuf)
        pltpu.sync_copy(rad_hbm.at[atoms, pl.ds(cid * R_HALF, R_HALF)], rbuf)
        pltpu.sync_copy(sbuf, acc_s.at[idx_v], add=True)
        pltpu.sync_copy(rbuf, acc_r.at[idx_v], add=True)

        @pl.when(cid == 0)
        def _():
            pltpu.sync_copy(comp_hbm.at[atoms], cbuf)
            pltpu.sync_copy(cbuf, acc_c.at[idx_v], add=True)

        return 0

    lax.fori_loop(0, CHUNKS_PER_TILE, chunk_step, 0)
    plsc.subcore_barrier()

    for b in range(ROWS_PER_TILE // CHUNK):
        rows = pl.ds(sid * ROWS_PER_TILE + b * CHUNK, CHUNK)
        pltpu.sync_copy(acc_s.at[rows], sbuf)
        pltpu.sync_copy(sbuf, out_s.at[cid, rows])
        pltpu.sync_copy(acc_r.at[rows], rbuf)
        pltpu.sync_copy(rbuf, out_r.at[cid, rows])

        @pl.when(cid == 0)
        def _():
            pltpu.sync_copy(acc_c.at[rows], cbuf)
            pltpu.sync_copy(cbuf, out_c.at[rows])


@jax.jit
def _segsum(comp, rad, spex, idx):
    mesh = plsc.VectorSubcoreMesh(
        core_axis_name="c", subcore_axis_name="s", num_cores=NC, num_subcores=NS
    )
    out_c, out_r, out_s = pl.kernel(
        _body,
        out_type=[
            jax.ShapeDtypeStruct((N_BUCKETS,), jnp.float32),
            jax.ShapeDtypeStruct((NC, N_BUCKETS, R_HALF), jnp.float32),
            jax.ShapeDtypeStruct((NC, N_BUCKETS, SG, 128), jnp.float32),
        ],
        mesh=mesh,
        compiler_params=pltpu.CompilerParams(use_tc_tiling_on_sc=False),
        scratch_types=[
            pltpu.VMEM_SHARED((N_BUCKETS,), jnp.float32),
            pltpu.VMEM_SHARED((N_BUCKETS, R_HALF), jnp.float32),
            pltpu.VMEM_SHARED((N_BUCKETS, SG, 128), jnp.float32),
            pltpu.VMEM((CHUNK,), jnp.float32),
            pltpu.VMEM((CHUNK, R_HALF), jnp.float32),
            pltpu.VMEM((CHUNK, SG, 128), jnp.float32),
            pltpu.VMEM((CHUNK,), jnp.int32),
        ],
    )(comp.reshape(N_ATOMS), rad, spex.reshape(N_ATOMS, C_SPEX // 128, 128), idx)

    rad_full = jnp.concatenate([out_r[0], out_r[1]], axis=1)
    spex_full = jnp.concatenate(
        [out_s[0].reshape(N_BUCKETS, SG * 128), out_s[1].reshape(N_BUCKETS, SG * 128)],
        axis=1,
    )
    return jnp.concatenate(
        [
            out_c.reshape(N_STRUCT, N_SPEC),
            rad_full.reshape(N_STRUCT, N_SPEC * C_RAD),
            spex_full.reshape(N_STRUCT, N_SPEC * C_SPEX),
        ],
        axis=1,
    )


def kernel(composition_features, radial_spectrum, spex_features, sum_indices):
    idx = sum_indices.astype(jnp.int32)
    return _segsum(composition_features, radial_spectrum, spex_features, idx)


# speedup vs baseline: 1.5384x; 1.3235x over previous
import jax
import jax.numpy as jnp
from jax import lax
from jax.experimental import pallas as pl
from jax.experimental.pallas import tpu as pltpu
from jax.experimental.pallas import tpu_sc as plsc

N_STRUCT = 512
N_SPEC = 4
N_BUCKETS = N_STRUCT * N_SPEC
N_ATOMS = 51200
C_COMP, C_RAD, C_SPEX = 1, 128, 1024

NC, NS, L = 2, 16, 16
CHUNK = 32
NBUF = 2
ATOMS_PER_TILE = N_ATOMS // NS
CHUNKS_PER_TILE = ATOMS_PER_TILE // CHUNK
S_HALF = C_SPEX // NC
R_HALF = C_RAD // NC
ROWS_PER_TILE = N_BUCKETS // NS


def _body(comp_hbm, rad_hbm, spex_hbm, idx_hbm,
          out_c, out_r, out_s,
          acc_r, acc_s, cstage,
          sbuf, rbuf, cbuf, idxv, acc_ct, cmerge, cvec,
          gsem0, gsem1, ssem0, ssem1):
    cid = lax.axis_index("c")
    sid = lax.axis_index("s")
    gsems = (gsem0, gsem1)
    ssems = (ssem0, ssem1)
    zeros = jnp.zeros((L,), jnp.float32)

    def zrow_s(i, _):
        for j in range(S_HALF // L):
            sbuf[0, i, pl.ds(j * L, L)] = zeros
        return 0

    lax.fori_loop(0, CHUNK, zrow_s, 0)

    def zrow_r(i, _):
        for j in range(R_HALF // L):
            rbuf[0, i, pl.ds(j * L, L)] = zeros
        return 0

    lax.fori_loop(0, CHUNK, zrow_r, 0)

    def zrow_c(i, _):
        acc_ct[pl.ds(i * L, L)] = zeros
        return 0

    lax.fori_loop(0, N_BUCKETS // L, zrow_c, 0)

    for b in range(ROWS_PER_TILE // CHUNK):
        rows = pl.ds(sid * ROWS_PER_TILE + b * CHUNK, CHUNK)
        pltpu.sync_copy(sbuf.at[0], acc_s.at[rows])
        pltpu.sync_copy(rbuf.at[0], acc_r.at[rows])
    plsc.subcore_barrier()

    base = sid * ATOMS_PER_TILE

    def gathers(k, b):
        atoms = pl.ds(base + k * CHUNK, CHUNK)
        g = gsems[b]
        a = pltpu.async_copy(idx_hbm.at[atoms], idxv.at[b], g)
        c = pltpu.async_copy(comp_hbm.at[atoms], cbuf.at[b], g)
        r = pltpu.async_copy(rad_hbm.at[atoms, pl.ds(cid * R_HALF, R_HALF)],
                             rbuf.at[b], g)
        s = pltpu.async_copy(spex_hbm.at[atoms, pl.ds(cid * S_HALF, S_HALF)],
                             sbuf.at[b], g)
        return (a, c, r, s)

    def scatters(b):
        s = ssems[b]
        x = pltpu.async_copy(sbuf.at[b], acc_s.at[idxv.at[b]], s, add=True)
        y = pltpu.async_copy(rbuf.at[b], acc_r.at[idxv.at[b]], s, add=True)
        return (x, y)

    for b in range(NBUF):
        for d in gathers(b, b):
            pass

    def pipe_step(g, _):
        for b in range(NBUF):
            k = g * NBUF + b
            atoms = pl.ds(base + k * CHUNK, CHUNK)
            pltpu.make_async_copy(idx_hbm.at[atoms], idxv.at[b], gsems[b]).wait()
            pltpu.make_async_copy(comp_hbm.at[atoms], cbuf.at[b], gsems[b]).wait()
            pltpu.make_async_copy(
                rad_hbm.at[atoms, pl.ds(cid * R_HALF, R_HALF)], rbuf.at[b],
                gsems[b]).wait()
            pltpu.make_async_copy(
                spex_hbm.at[atoms, pl.ds(cid * S_HALF, S_HALF)], sbuf.at[b],
                gsems[b]).wait()
            for v in range(CHUNK // L):
                iv = idxv[b, pl.ds(v * L, L)]
                cv = cbuf[b, pl.ds(v * L, L)]
                plsc.addupdate_scatter(acc_ct, [iv], cv)
            scatters(b)
            pltpu.make_async_copy(sbuf.at[b], acc_s.at[idxv.at[b]], ssems[b]).wait()
            pltpu.make_async_copy(rbuf.at[b], acc_r.at[idxv.at[b]], ssems[b]).wait()

            @pl.when(k + NBUF < CHUNKS_PER_TILE)
            def _():
                gathers(k + NBUF, b)

        return 0

    lax.fori_loop(0, CHUNKS_PER_TILE // NBUF, pipe_step, 0)

    pltpu.sync_copy(acc_ct, cstage.at[sid])
    plsc.subcore_barrier()

    for b in range(ROWS_PER_TILE // CHUNK):
        rows = pl.ds(sid * ROWS_PER_TILE + b * CHUNK, CHUNK)
        pltpu.sync_copy(acc_s.at[rows], sbuf.at[0])
        pltpu.sync_copy(sbuf.at[0], out_s.at[cid, rows])
        pltpu.sync_copy(acc_r.at[rows], rbuf.at[0])
        pltpu.sync_copy(rbuf.at[0], out_r.at[cid, rows])

    @pl.when(cid == 0)
    def _():
        pltpu.sync_copy(cstage.at[:, pl.ds(sid * ROWS_PER_TILE, ROWS_PER_TILE)],
                        cmerge)
        for j in range(ROWS_PER_TILE // L):
            acc = zeros
            for t in range(NS):
                acc = acc + cmerge[t, pl.ds(j * L, L)]
            cvec[pl.ds(j * L, L)] = acc
        pltpu.sync_copy(cvec, out_c.at[pl.ds(sid * ROWS_PER_TILE, ROWS_PER_TILE)])


@jax.jit
def _segsum(comp, rad, spex, idx):
    mesh = plsc.VectorSubcoreMesh(
        core_axis_name="c", subcore_axis_name="s", num_cores=NC, num_subcores=NS
    )
    out_c, out_r, out_s = pl.kernel(
        _body,
        out_type=[
            jax.ShapeDtypeStruct((N_BUCKETS,), jnp.float32),
            jax.ShapeDtypeStruct((NC, N_BUCKETS, R_HALF), jnp.float32),
            jax.ShapeDtypeStruct((NC, N_BUCKETS, S_HALF), jnp.float32),
        ],
        mesh=mesh,
        compiler_params=pltpu.CompilerParams(
            use_tc_tiling_on_sc=False, needs_layout_passes=False
        ),
        scratch_types=[
            pltpu.VMEM_SHARED((N_BUCKETS, R_HALF), jnp.float32),
            pltpu.VMEM_SHARED((N_BUCKETS, S_HALF), jnp.float32),
            pltpu.VMEM_SHARED((NS, N_BUCKETS), jnp.float32),
            pltpu.VMEM((NBUF, CHUNK, S_HALF), jnp.float32),
            pltpu.VMEM((NBUF, CHUNK, R_HALF), jnp.float32),
            pltpu.VMEM((NBUF, CHUNK), jnp.float32),
            pltpu.VMEM((NBUF, CHUNK), jnp.int32),
            pltpu.VMEM((N_BUCKETS,), jnp.float32),
            pltpu.VMEM((NS, ROWS_PER_TILE), jnp.float32),
            pltpu.VMEM((ROWS_PER_TILE,), jnp.float32),
            pltpu.SemaphoreType.DMA,
            pltpu.SemaphoreType.DMA,
            pltpu.SemaphoreType.DMA,
            pltpu.SemaphoreType.DMA,
        ],
    )(comp.reshape(N_ATOMS), rad, spex, idx)

    rad_full = jnp.concatenate([out_r[0], out_r[1]], axis=1)
    spex_full = jnp.concatenate([out_s[0], out_s[1]], axis=1)
    return jnp.concatenate(
        [
            out_c.reshape(N_STRUCT, N_SPEC),
            rad_full.reshape(N_STRUCT, N_SPEC * C_RAD),
            spex_full.reshape(N_STRUCT, N_SPEC * C_SPEX),
        ],
        axis=1,
    )


def kernel(composition_features, radial_spectrum, spex_features, sum_indices):
    idx = sum_indices.astype(jnp.int32)
    return _segsum(composition_features, radial_spectrum, spex_features, idx)


# speedup vs baseline: 1.5841x; 1.0297x over previous
import jax
import jax.numpy as jnp
from jax import lax
from jax.experimental import pallas as pl
from jax.experimental.pallas import tpu as pltpu
from jax.experimental.pallas import tpu_sc as plsc

N_STRUCT = 512
N_SPEC = 4
N_BUCKETS = N_STRUCT * N_SPEC
N_ATOMS = 51200
C_COMP, C_RAD, C_SPEX = 1, 128, 1024

NC, NS, L = 2, 16, 16
CHUNK = 32
NBUF = 2
ATOMS_PER_TILE = N_ATOMS // NS
CHUNKS_PER_TILE = ATOMS_PER_TILE // CHUNK
S_HALF = C_SPEX // NC
R_HALF = C_RAD
ROWS_PER_TILE = N_BUCKETS // NS


def _body(comp_hbm, rad_hbm, spex_hbm, idx_hbm,
          out_c, out_r, out_s,
          acc_r, acc_s, cstage,
          sbuf, rbuf, cbuf, idxv, acc_ct, cmerge, cvec,
          gsem0, gsem1, ssem0, ssem1):
    cid = lax.axis_index("c")
    sid = lax.axis_index("s")
    gsems = (gsem0, gsem1)
    ssems = (ssem0, ssem1)
    zeros = jnp.zeros((L,), jnp.float32)

    def zrow_s(i, _):
        for j in range(S_HALF // L):
            sbuf[0, i, pl.ds(j * L, L)] = zeros
        return 0

    lax.fori_loop(0, CHUNK, zrow_s, 0)

    def zrow_r(i, _):
        for j in range(R_HALF // L):
            rbuf[0, i, pl.ds(j * L, L)] = zeros
        return 0

    lax.fori_loop(0, CHUNK, zrow_r, 0)

    def zrow_c(i, _):
        acc_ct[pl.ds(i * L, L)] = zeros
        return 0

    lax.fori_loop(0, N_BUCKETS // L, zrow_c, 0)

    for b in range(ROWS_PER_TILE // CHUNK):
        rows = pl.ds(sid * ROWS_PER_TILE + b * CHUNK, CHUNK)
        pltpu.sync_copy(sbuf.at[0], acc_s.at[rows])
        pltpu.sync_copy(rbuf.at[0], acc_r.at[rows])
    plsc.subcore_barrier()

    base = sid * ATOMS_PER_TILE

    def gathers(k, b):
        atoms = pl.ds(base + k * CHUNK, CHUNK)
        g = gsems[b]
        pltpu.async_copy(idx_hbm.at[atoms], idxv.at[b], g)
        pltpu.async_copy(comp_hbm.at[atoms], cbuf.at[b], g)
        pltpu.async_copy(spex_hbm.at[atoms, pl.ds(cid * S_HALF, S_HALF)],
                         sbuf.at[b], g)

        @pl.when(cid == 1)
        def _():
            pltpu.async_copy(rad_hbm.at[atoms], rbuf.at[b], g)

    def scatters(b):
        s = ssems[b]
        pltpu.async_copy(sbuf.at[b], acc_s.at[idxv.at[b]], s, add=True)

        @pl.when(cid == 1)
        def _():
            pltpu.async_copy(rbuf.at[b], acc_r.at[idxv.at[b]], s, add=True)

    for b in range(NBUF):
        gathers(b, b)

    def pipe_step(g, _):
        for b in range(NBUF):
            k = g * NBUF + b
            atoms = pl.ds(base + k * CHUNK, CHUNK)
            pltpu.make_async_copy(idx_hbm.at[atoms], idxv.at[b], gsems[b]).wait()
            pltpu.make_async_copy(comp_hbm.at[atoms], cbuf.at[b], gsems[b]).wait()
            pltpu.make_async_copy(
                spex_hbm.at[atoms, pl.ds(cid * S_HALF, S_HALF)], sbuf.at[b],
                gsems[b]).wait()

            @pl.when(cid == 1)
            def _():
                pltpu.make_async_copy(rad_hbm.at[atoms], rbuf.at[b],
                                      gsems[b]).wait()
            for v in range(CHUNK // L):
                iv = idxv[b, pl.ds(v * L, L)]
                cv = cbuf[b, pl.ds(v * L, L)]
                plsc.addupdate_scatter(acc_ct, [iv], cv)
            scatters(b)
            pltpu.make_async_copy(sbuf.at[b], acc_s.at[idxv.at[b]], ssems[b]).wait()

            @pl.when(cid == 1)
            def _():
                pltpu.make_async_copy(rbuf.at[b], acc_r.at[idxv.at[b]],
                                      ssems[b]).wait()

            @pl.when(k + NBUF < CHUNKS_PER_TILE)
            def _():
                gathers(k + NBUF, b)

        return 0

    lax.fori_loop(0, CHUNKS_PER_TILE // NBUF, pipe_step, 0)

    pltpu.sync_copy(acc_ct, cstage.at[sid])
    plsc.subcore_barrier()

    for b in range(ROWS_PER_TILE // CHUNK):
        rows = pl.ds(sid * ROWS_PER_TILE + b * CHUNK, CHUNK)
        pltpu.sync_copy(acc_s.at[rows], sbuf.at[0])
        pltpu.sync_copy(sbuf.at[0], out_s.at[rows, pl.ds(cid * S_HALF, S_HALF)])

        @pl.when(cid == 1)
        def _():
            pltpu.sync_copy(acc_r.at[rows], rbuf.at[0])
            pltpu.sync_copy(rbuf.at[0], out_r.at[rows])

    @pl.when(cid == 0)
    def _():
        pltpu.sync_copy(cstage.at[:, pl.ds(sid * ROWS_PER_TILE, ROWS_PER_TILE)],
                        cmerge)
        for j in range(ROWS_PER_TILE // L):
            acc = zeros
            for t in range(NS):
                acc = acc + cmerge[t, pl.ds(j * L, L)]
            cvec[pl.ds(j * L, L)] = acc
        pltpu.sync_copy(cvec, out_c.at[pl.ds(sid * ROWS_PER_TILE, ROWS_PER_TILE)])


@jax.jit
def _segsum(comp, rad, spex, idx):
    mesh = plsc.VectorSubcoreMesh(
        core_axis_name="c", subcore_axis_name="s", num_cores=NC, num_subcores=NS
    )
    out_c, out_r, out_s = pl.kernel(
        _body,
        out_type=[
            jax.ShapeDtypeStruct((N_BUCKETS,), jnp.float32),
            jax.ShapeDtypeStruct((N_BUCKETS, R_HALF), jnp.float32),
            jax.ShapeDtypeStruct((N_BUCKETS, C_SPEX), jnp.float32),
        ],
        mesh=mesh,
        compiler_params=pltpu.CompilerParams(
            use_tc_tiling_on_sc=False, needs_layout_passes=False
        ),
        scratch_types=[
            pltpu.VMEM_SHARED((N_BUCKETS, R_HALF), jnp.float32),
            pltpu.VMEM_SHARED((N_BUCKETS, S_HALF), jnp.float32),
            pltpu.VMEM_SHARED((NS, N_BUCKETS), jnp.float32),
            pltpu.VMEM((NBUF, CHUNK, S_HALF), jnp.float32),
            pltpu.VMEM((NBUF, CHUNK, R_HALF), jnp.float32),
            pltpu.VMEM((NBUF, CHUNK), jnp.float32),
            pltpu.VMEM((NBUF, CHUNK), jnp.int32),
            pltpu.VMEM((N_BUCKETS,), jnp.float32),
            pltpu.VMEM((NS, ROWS_PER_TILE), jnp.float32),
            pltpu.VMEM((ROWS_PER_TILE,), jnp.float32),
            pltpu.SemaphoreType.DMA,
            pltpu.SemaphoreType.DMA,
            pltpu.SemaphoreType.DMA,
            pltpu.SemaphoreType.DMA,
        ],
    )(comp.reshape(N_ATOMS), rad, spex, idx)

    return jnp.concatenate(
        [
            out_c.reshape(N_STRUCT, N_SPEC),
            out_r.reshape(N_STRUCT, N_SPEC * C_RAD),
            out_s.reshape(N_STRUCT, N_SPEC * C_SPEX),
        ],
        axis=1,
    )


def kernel(composition_features, radial_spectrum, spex_features, sum_indices):
    idx = sum_indices.astype(jnp.int32)
    return _segsum(composition_features, radial_spectrum, spex_features, idx)


# speedup vs baseline: 2.4150x; 1.5245x over previous
import jax
import jax.numpy as jnp
from jax import lax
from jax.experimental import pallas as pl
from jax.experimental.pallas import tpu as pltpu
from jax.experimental.pallas import tpu_sc as plsc

N_STRUCT = 512
N_SPEC = 4
N_BUCKETS = N_STRUCT * N_SPEC
N_ATOMS = 51200
C_COMP, C_RAD, C_SPEX = 1, 128, 1024

NC, NS, L = 2, 16, 16
CHUNK = 32
NBUF = 2
SEGS = CHUNK * 4
ATOMS_PER_TILE = N_ATOMS // NS
CHUNKS_PER_TILE = ATOMS_PER_TILE // CHUNK
N_SROWS = N_BUCKETS * 4
ROWS_PER_TILE = N_BUCKETS // NS


def _body(comp_hbm, rad_hbm, spex_hbm, idx_hbm,
          out_c, out_r, out_s,
          acc_r, acc_s, cstage,
          sbuf, rbuf, cbuf, idxv0, idxv1, ix0, ix1, acc_ct, cmerge, cvec,
          gsem0, gsem1, ssem0, ssem1):
    cid = lax.axis_index("c")
    sid = lax.axis_index("s")
    gsems = (gsem0, gsem1)
    ssems = (ssem0, ssem1)
    idxvs = (idxv0, idxv1)
    ixs = (ix0, ix1)
    zeros = jnp.zeros((L,), jnp.float32)
    iota = lax.iota(jnp.int32, L)

    def zrow_s(i, _):
        for j in range(128 // L):
            sbuf[0, i, pl.ds(j * L, L)] = zeros
        return 0

    lax.fori_loop(0, SEGS, zrow_s, 0)

    def zrow_r(i, _):
        for j in range(C_RAD // L):
            rbuf[0, i, pl.ds(j * L, L)] = zeros
        return 0

    lax.fori_loop(0, CHUNK, zrow_r, 0)

    def zrow_c(i, _):
        acc_ct[pl.ds(i * L, L)] = zeros
        return 0

    lax.fori_loop(0, N_BUCKETS // L, zrow_c, 0)

    srows_per_tile = N_SROWS // NS
    for e in range(srows_per_tile // SEGS):
        pltpu.sync_copy(sbuf.at[0],
                        acc_s.at[pl.ds(sid * srows_per_tile + e * SEGS, SEGS)])
    for e in range(ROWS_PER_TILE // CHUNK):
        pltpu.sync_copy(rbuf.at[0],
                        acc_r.at[pl.ds(sid * ROWS_PER_TILE + e * CHUNK, CHUNK)])
    plsc.subcore_barrier()

    base = sid * ATOMS_PER_TILE

    def gathers(k, b):
        atoms = pl.ds(base + k * CHUNK, CHUNK)
        g = gsems[b]
        pltpu.async_copy(idx_hbm.at[atoms], idxvs[b], g)
        pltpu.async_copy(comp_hbm.at[atoms], cbuf.at[b], g)
        rt0 = (base + k * CHUNK) // 8
        for tr in range(4):
            q0 = (rt0 + tr) * 64 + cid * 32
            pltpu.async_copy(spex_hbm.at[pl.ds(q0, 32), :],
                             sbuf.at[b, pl.ds(tr * 32, 32), :], g)

        @pl.when(cid == 1)
        def _():
            pltpu.async_copy(rad_hbm.at[atoms], rbuf.at[b], g)

    for b in range(NBUF):
        gathers(b, b)

    def pipe_step(g, _):
        for b in range(NBUF):
            k = g * NBUF + b
            atoms = pl.ds(base + k * CHUNK, CHUNK)
            rt0 = (base + k * CHUNK) // 8
            pltpu.make_async_copy(idx_hbm.at[atoms], idxvs[b], gsems[b]).wait()
            pltpu.make_async_copy(comp_hbm.at[atoms], cbuf.at[b], gsems[b]).wait()
            for tr in range(4):
                q0 = (rt0 + tr) * 64 + cid * 32
                pltpu.make_async_copy(spex_hbm.at[pl.ds(q0, 32), :],
                                      sbuf.at[b, pl.ds(tr * 32, 32), :],
                                      gsems[b]).wait()

            @pl.when(cid == 1)
            def _():
                pltpu.make_async_copy(rad_hbm.at[atoms], rbuf.at[b],
                                      gsems[b]).wait()

            for blk in range(SEGS // L):
                p = iota + blk * L
                a_rel = 8 * (p >> 5) + (p & 7)
                gl = (p >> 3) & 3
                iv = plsc.load_gather(idxvs[b], [a_rel])
                ixs[b][pl.ds(blk * L, L)] = iv * 4 + gl

            for v in range(CHUNK // L):
                ia = idxvs[b][pl.ds(v * L, L)]
                cv = cbuf[b, pl.ds(v * L, L)]
                plsc.addupdate_scatter(acc_ct, [ia], cv)

            pltpu.async_copy(sbuf.at[b], acc_s.at[ixs[b]], ssems[b], add=True)

            @pl.when(cid == 1)
            def _():
                pltpu.async_copy(rbuf.at[b], acc_r.at[idxvs[b]], ssems[b],
                                 add=True)

            pltpu.make_async_copy(sbuf.at[b], acc_s.at[ixs[b]], ssems[b]).wait()

            @pl.when(cid == 1)
            def _():
                pltpu.make_async_copy(rbuf.at[b], acc_r.at[idxvs[b]],
                                      ssems[b]).wait()

            @pl.when(k + NBUF < CHUNKS_PER_TILE)
            def _():
                gathers(k + NBUF, b)

        return 0

    lax.fori_loop(0, CHUNKS_PER_TILE // NBUF, pipe_step, 0)

    pltpu.sync_copy(acc_ct, cstage.at[sid])
    plsc.subcore_barrier()

    for e in range(srows_per_tile // SEGS):
        rows = pl.ds(sid * srows_per_tile + e * SEGS, SEGS)
        pltpu.sync_copy(acc_s.at[rows], sbuf.at[0])
        pltpu.sync_copy(sbuf.at[0], out_s.at[cid, rows])

    @pl.when(cid == 1)
    def _():
        for e in range(ROWS_PER_TILE // CHUNK):
            rows = pl.ds(sid * ROWS_PER_TILE + e * CHUNK, CHUNK)
            pltpu.sync_copy(acc_r.at[rows], rbuf.at[0])
            pltpu.sync_copy(rbuf.at[0], out_r.at[rows])

    @pl.when(cid == 0)
    def _():
        pltpu.sync_copy(cstage.at[:, pl.ds(sid * ROWS_PER_TILE, ROWS_PER_TILE)],
                        cmerge)
        for j in range(ROWS_PER_TILE // L):
            acc = zeros
            for t in range(NS):
                acc = acc + cmerge[t, pl.ds(j * L, L)]
            cvec[pl.ds(j * L, L)] = acc
        pltpu.sync_copy(cvec, out_c.at[pl.ds(sid * ROWS_PER_TILE, ROWS_PER_TILE)])


@jax.jit
def _segsum(comp, rad, spex, idx):
    mesh = plsc.VectorSubcoreMesh(
        core_axis_name="c", subcore_axis_name="s", num_cores=NC, num_subcores=NS
    )
    spex_p = (
        spex.reshape(N_ATOMS // 8, 8, 8, 128)
        .transpose(0, 2, 1, 3)
        .reshape(N_ATOMS * 8, 128)
    )
    out_c, out_r, out_s = pl.kernel(
        _body,
        out_type=[
            jax.ShapeDtypeStruct((N_BUCKETS,), jnp.float32),
            jax.ShapeDtypeStruct((N_BUCKETS, C_RAD), jnp.float32),
            jax.ShapeDtypeStruct((NC, N_SROWS, 128), jnp.float32),
        ],
        mesh=mesh,
        compiler_params=pltpu.CompilerParams(
            use_tc_tiling_on_sc=False, needs_layout_passes=False
        ),
        scratch_types=[
            pltpu.VMEM_SHARED((N_BUCKETS, C_RAD), jnp.float32),
            pltpu.VMEM_SHARED((N_SROWS, 128), jnp.float32),
            pltpu.VMEM_SHARED((NS, N_BUCKETS), jnp.float32),
            pltpu.VMEM((NBUF, SEGS, 128), jnp.float32),
            pltpu.VMEM((NBUF, CHUNK, C_RAD), jnp.float32),
            pltpu.VMEM((NBUF, CHUNK), jnp.float32),
            pltpu.VMEM((CHUNK,), jnp.int32),
            pltpu.VMEM((CHUNK,), jnp.int32),
            pltpu.VMEM((SEGS,), jnp.int32),
            pltpu.VMEM((SEGS,), jnp.int32),
            pltpu.VMEM((N_BUCKETS,), jnp.float32),
            pltpu.VMEM((NS, ROWS_PER_TILE), jnp.float32),
            pltpu.VMEM((ROWS_PER_TILE,), jnp.float32),
            pltpu.SemaphoreType.DMA,
            pltpu.SemaphoreType.DMA,
            pltpu.SemaphoreType.DMA,
            pltpu.SemaphoreType.DMA,
        ],
    )(comp.reshape(N_ATOMS), rad, spex_p, idx)

    spex_full = jnp.concatenate(
        [out_s[0].reshape(N_BUCKETS, 512), out_s[1].reshape(N_BUCKETS, 512)],
        axis=1,
    )
    return jnp.concatenate(
        [
            out_c.reshape(N_STRUCT, N_SPEC),
            out_r.reshape(N_STRUCT, N_SPEC * C_RAD),
            spex_full.reshape(N_STRUCT, N_SPEC * C_SPEX),
        ],
        axis=1,
    )


def kernel(composition_features, radial_spectrum, spex_features, sum_indices):
    idx = sum_indices.astype(jnp.int32)
    return _segsum(composition_features, radial_spectrum, spex_features, idx)


# speedup vs baseline: 2.7706x; 1.1472x over previous
import jax
import jax.numpy as jnp
from jax import lax
from jax.experimental import pallas as pl
from jax.experimental.pallas import tpu as pltpu
from jax.experimental.pallas import tpu_sc as plsc

N_STRUCT = 512
N_SPEC = 4
N_BUCKETS = N_STRUCT * N_SPEC
N_ATOMS = 51200
C_COMP, C_RAD, C_SPEX = 1, 128, 1024

NC, NS, L = 2, 16, 16
CHUNK = 32
NBUF = 2
SEGS = CHUNK * 4
ATOMS_PER_TILE = N_ATOMS // NS
CHUNKS_PER_TILE = ATOMS_PER_TILE // CHUNK
N_SROWS = N_BUCKETS * 4
ROWS_PER_TILE = N_BUCKETS // NS


def _body(comp_hbm, rad_hbm, spex_hbm, idx_hbm,
          out_c, out_r, out_s,
          acc_r, acc_s, cstage,
          sbuf, rbuf, cbuf, idxv0, idxv1, ix0, ix1, acc_ct, cmerge, cvec,
          gsem0, gsem1, ssem0, ssem1):
    cid = lax.axis_index("c")
    sid = lax.axis_index("s")
    gsems = (gsem0, gsem1)
    ssems = (ssem0, ssem1)
    idxvs = (idxv0, idxv1)
    ixs = (ix0, ix1)
    zeros = jnp.zeros((L,), jnp.float32)
    iota = lax.iota(jnp.int32, L)

    def zrow_s(i, _):
        for j in range(128 // L):
            sbuf[0, i, pl.ds(j * L, L)] = zeros
        return 0

    lax.fori_loop(0, SEGS, zrow_s, 0)

    def zrow_r(i, _):
        for j in range(64 // L):
            rbuf[0, i, pl.ds(j * L, L)] = zeros
            rbuf[1, i, pl.ds(j * L, L)] = zeros
        return 0

    lax.fori_loop(0, CHUNK, zrow_r, 0)

    def zrow_c(i, _):
        acc_ct[pl.ds(i * L, L)] = zeros
        return 0

    lax.fori_loop(0, N_BUCKETS // L, zrow_c, 0)

    srows_per_tile = N_SROWS // NS
    for e in range(srows_per_tile // SEGS):
        pltpu.sync_copy(sbuf.at[0],
                        acc_s.at[pl.ds(sid * srows_per_tile + e * SEGS, SEGS)])
    for e in range(ROWS_PER_TILE // CHUNK):
        pltpu.sync_copy(rbuf.at[e % NBUF],
                        acc_r.at[pl.ds(sid * ROWS_PER_TILE + e * CHUNK, CHUNK)])
    plsc.subcore_barrier()

    base = sid * ATOMS_PER_TILE

    def gathers(k, b):
        atoms = pl.ds(base + k * CHUNK, CHUNK)
        g = gsems[b]
        pltpu.async_copy(idx_hbm.at[atoms], idxvs[b], g)
        pltpu.async_copy(comp_hbm.at[atoms], cbuf.at[b], g)
        rt0 = (base + k * CHUNK) // 8
        for tr in range(4):
            q0 = (rt0 + tr) * 64 + cid * 32
            pltpu.async_copy(spex_hbm.at[pl.ds(q0, 32), :],
                             sbuf.at[b, pl.ds(tr * 32, 32), :], g)

        pltpu.async_copy(rad_hbm.at[atoms, pl.ds(cid * 64, 64)], rbuf.at[b], g)

    for b in range(NBUF):
        gathers(b, b)

    def pipe_step(g, _):
        for b in range(NBUF):
            k = g * NBUF + b
            atoms = pl.ds(base + k * CHUNK, CHUNK)
            rt0 = (base + k * CHUNK) // 8
            pltpu.make_async_copy(idx_hbm.at[atoms], idxvs[b], gsems[b]).wait()
            pltpu.make_async_copy(comp_hbm.at[atoms], cbuf.at[b], gsems[b]).wait()
            for tr in range(4):
                q0 = (rt0 + tr) * 64 + cid * 32
                pltpu.make_async_copy(spex_hbm.at[pl.ds(q0, 32), :],
                                      sbuf.at[b, pl.ds(tr * 32, 32), :],
                                      gsems[b]).wait()

            pltpu.make_async_copy(rad_hbm.at[atoms, pl.ds(cid * 64, 64)],
                                  rbuf.at[b], gsems[b]).wait()

            for blk in range(SEGS // L):
                p = iota + blk * L
                a_rel = 8 * (p >> 5) + (p & 7)
                gl = (p >> 3) & 3
                iv = plsc.load_gather(idxvs[b], [a_rel])
                ixs[b][pl.ds(blk * L, L)] = iv + gl * N_BUCKETS

            for v in range(CHUNK // L):
                ia = idxvs[b][pl.ds(v * L, L)]
                cv = cbuf[b, pl.ds(v * L, L)]
                plsc.addupdate_scatter(acc_ct, [ia], cv)

            pltpu.async_copy(sbuf.at[b], acc_s.at[ixs[b]], ssems[b], add=True)

            pltpu.async_copy(rbuf.at[b], acc_r.at[idxvs[b]], ssems[b],
                             add=True)

            pltpu.make_async_copy(sbuf.at[b], acc_s.at[ixs[b]], ssems[b]).wait()

            pltpu.make_async_copy(rbuf.at[b], acc_r.at[idxvs[b]],
                                  ssems[b]).wait()

            @pl.when(k + NBUF < CHUNKS_PER_TILE)
            def _():
                gathers(k + NBUF, b)

        return 0

    lax.fori_loop(0, CHUNKS_PER_TILE // NBUF, pipe_step, 0)

    pltpu.sync_copy(acc_ct, cstage.at[sid])
    plsc.subcore_barrier()

    for gblk in range(4):
        rows = pl.ds(gblk * N_BUCKETS + sid * ROWS_PER_TILE, ROWS_PER_TILE)
        pltpu.sync_copy(acc_s.at[rows], sbuf.at[0])
        pltpu.sync_copy(
            sbuf.at[0],
            out_s.at[pl.ds(sid * ROWS_PER_TILE, ROWS_PER_TILE),
                     pl.ds((4 * cid + gblk) * 128, 128)])
    for e in range(ROWS_PER_TILE // CHUNK):
        rows = pl.ds(sid * ROWS_PER_TILE + e * CHUNK, CHUNK)
        pltpu.sync_copy(acc_r.at[rows], rbuf.at[e % NBUF])
        pltpu.sync_copy(rbuf.at[e % NBUF], out_r.at[cid, rows])

    @pl.when(cid == 0)
    def _():
        pltpu.sync_copy(cstage.at[:, pl.ds(sid * ROWS_PER_TILE, ROWS_PER_TILE)],
                        cmerge)
        for j in range(ROWS_PER_TILE // L):
            acc = zeros
            for t in range(NS):
                acc = acc + cmerge[t, pl.ds(j * L, L)]
            cvec[pl.ds(j * L, L)] = acc
        pltpu.sync_copy(cvec, out_c.at[pl.ds(sid * ROWS_PER_TILE, ROWS_PER_TILE)])


@jax.jit
def _segsum(comp, rad, spex, idx):
    mesh = plsc.VectorSubcoreMesh(
        core_axis_name="c", subcore_axis_name="s", num_cores=NC, num_subcores=NS
    )
    spex_p = (
        spex.reshape(N_ATOMS // 8, 8, 8, 128)
        .transpose(0, 2, 1, 3)
        .reshape(N_ATOMS * 8, 128)
    )
    out_c, out_r, out_s = pl.kernel(
        _body,
        out_type=[
            jax.ShapeDtypeStruct((N_BUCKETS,), jnp.float32),
            jax.ShapeDtypeStruct((NC, N_BUCKETS, 64), jnp.float32),
            jax.ShapeDtypeStruct((N_BUCKETS, C_SPEX), jnp.float32),
        ],
        mesh=mesh,
        compiler_params=pltpu.CompilerParams(
            use_tc_tiling_on_sc=False, needs_layout_passes=False
        ),
        scratch_types=[
            pltpu.VMEM_SHARED((N_BUCKETS, 64), jnp.float32),
            pltpu.VMEM_SHARED((N_SROWS, 128), jnp.float32),
            pltpu.VMEM_SHARED((NS, N_BUCKETS), jnp.float32),
            pltpu.VMEM((NBUF, SEGS, 128), jnp.float32),
            pltpu.VMEM((NBUF, CHUNK, 64), jnp.float32),
            pltpu.VMEM((NBUF, CHUNK), jnp.float32),
            pltpu.VMEM((CHUNK,), jnp.int32),
            pltpu.VMEM((CHUNK,), jnp.int32),
            pltpu.VMEM((SEGS,), jnp.int32),
            pltpu.VMEM((SEGS,), jnp.int32),
            pltpu.VMEM((N_BUCKETS,), jnp.float32),
            pltpu.VMEM((NS, ROWS_PER_TILE), jnp.float32),
            pltpu.VMEM((ROWS_PER_TILE,), jnp.float32),
            pltpu.SemaphoreType.DMA,
            pltpu.SemaphoreType.DMA,
            pltpu.SemaphoreType.DMA,
            pltpu.SemaphoreType.DMA,
        ],
    )(comp.reshape(N_ATOMS), rad, spex_p, idx)

    rad_full = jnp.concatenate([out_r[0], out_r[1]], axis=1)
    return jnp.concatenate(
        [
            out_c.reshape(N_STRUCT, N_SPEC),
            rad_full.reshape(N_STRUCT, N_SPEC * C_RAD),
            out_s.reshape(N_STRUCT, N_SPEC * C_SPEX),
        ],
        axis=1,
    )


def kernel(composition_features, radial_spectrum, spex_features, sum_indices):
    idx = sum_indices.astype(jnp.int32)
    return _segsum(composition_features, radial_spectrum, spex_features, idx)
